# baseline (speedup 1.0000x reference)
import functools

import jax
import jax.numpy as jnp
from jax import lax
from jax.experimental import pallas as pl
from jax.experimental.pallas import tpu as pltpu
from jax.experimental.pallas import tpu_sc as plsc

B = 16384
D = 64
L = 16
C = 128
N = 1000000
U = 512
NUNITS = 1952
TAIL = N - NUNITS * U
EPS = 1e-12


def _transpose_table(node_t):
    info = plsc.get_sparse_core_info()
    nw = info.num_cores * info.num_subcores
    upw = NUNITS // nw

    mesh = plsc.VectorSubcoreMesh(core_axis_name="c", subcore_axis_name="s")

    @functools.partial(
        pl.kernel,
        mesh=mesh,
        out_type=jax.ShapeDtypeStruct((N, D), jnp.float32),
        compiler_params=pltpu.CompilerParams(needs_layout_passes=False),
        scratch_types=[
            pltpu.VMEM((D, TAIL), jnp.float32),
            pltpu.VMEM((TAIL, D), jnp.float32),
        ],
    )
    def transpose(nt_hbm, out_hbm, in_v, out_v):
        wid = lax.axis_index("s") * info.num_cores + lax.axis_index("c")

        rows = [lax.iota(jnp.int32, L) + 16 * m for m in range(D // L)]

        def do_unit(base, w):
            pltpu.sync_copy(nt_hbm.at[:, pl.ds(base, w)],
                            in_v.at[:, pl.ds(0, w)])

            @plsc.parallel_loop(0, w, unroll=4)
            def trow(r):
                col = jnp.broadcast_to(r, (L,)).astype(jnp.int32)
                for m in range(D // L):
                    out_v[r, pl.ds(16 * m, L)] = plsc.load_gather(
                        in_v, [rows[m], col])

            pltpu.sync_copy(out_v.at[pl.ds(0, w), :],
                            out_hbm.at[pl.ds(base, w), :])

        def unit(u_local, carry):
            do_unit((wid * upw + u_local) * U, U)
            return carry

        lax.fori_loop(0, upw, unit, 0)

        @pl.when(wid == 0)
        def _tail():
            do_unit(NUNITS * U, TAIL)

    return transpose(node_t)


def kernel(head_index, rel_type, tail_index, node_emb, rel_emb):
    info = plsc.get_sparse_core_info()
    nw = info.num_cores * info.num_subcores
    bpw = B // nw

    node_rm = _transpose_table(node_emb.T)

    mesh = plsc.VectorSubcoreMesh(core_axis_name="c", subcore_axis_name="s")

    @functools.partial(
        pl.kernel,
        mesh=mesh,
        out_type=jax.ShapeDtypeStruct((B, D), jnp.float32),
        scratch_types=[
            pltpu.VMEM((bpw,), jnp.int32),
            pltpu.VMEM((bpw,), jnp.int32),
            pltpu.VMEM((bpw,), jnp.int32),
            pltpu.VMEM((C, D), jnp.float32),
            pltpu.VMEM((C, D), jnp.float32),
            pltpu.VMEM((C, D), jnp.float32),
            pltpu.VMEM((C, D), jnp.float32),
            pltpu.SemaphoreType.DMA,
        ],
    )
    def trans_e(h_idx_hbm, r_idx_hbm, t_idx_hbm, node_hbm, rel_hbm, out_hbm,
                hi_v, ri_v, ti_v, h_v, r_v, t_v, o_v, sem):
        wid = lax.axis_index("s") * info.num_cores + lax.axis_index("c")
        base = wid * bpw

        pltpu.sync_copy(h_idx_hbm.at[pl.ds(base, bpw)], hi_v)
        pltpu.sync_copy(r_idx_hbm.at[pl.ds(base, bpw)], ri_v)
        pltpu.sync_copy(t_idx_hbm.at[pl.ds(base, bpw)], ti_v)

        iota = lax.iota(jnp.int32, L)
        perms = [iota ^ sh for sh in (1, 2, 4, 8)]
        gdn = lax.GatherDimensionNumbers(
            offset_dims=(), collapsed_slice_dims=(0,), start_index_map=(0,))

        def lane_total(v):
            for p in perms:
                v = v + lax.gather(
                    v, p[:, None], dimension_numbers=gdn, slice_sizes=(1,),
                    mode=lax.GatherScatterMode.PROMISE_IN_BOUNDS)
            return v

        def chunk(ci, carry):
            cbase = ci * C
            copies = []
            for jj in range(C // L):
                hv = hi_v[pl.ds(cbase + jj * L, L)]
                tv = ti_v[pl.ds(cbase + jj * L, L)]
                rv = ri_v[pl.ds(cbase + jj * L, L)]
                for k in range(L):
                    r = jj * L + k
                    copies.append(pltpu.async_copy(
                        node_hbm.at[hv[k]], h_v.at[r], sem))
                    copies.append(pltpu.async_copy(
                        node_hbm.at[tv[k]], t_v.at[r], sem))
                    copies.append(pltpu.async_copy(
                        rel_hbm.at[rv[k]], r_v.at[r], sem))
            for cp in copies:
                cp.wait()

            def row(i, carry2):
                hs = [h_v[i, pl.ds(c * L, L)] for c in range(D // L)]
                ts = [t_v[i, pl.ds(c * L, L)] for c in range(D // L)]
                rs = [r_v[i, pl.ds(c * L, L)] for c in range(D // L)]

                ah = (jnp.abs(hs[0]) + jnp.abs(hs[1])) + (jnp.abs(hs[2]) + jnp.abs(hs[3]))
                at = (jnp.abs(ts[0]) + jnp.abs(ts[1])) + (jnp.abs(ts[2]) + jnp.abs(ts[3]))
                nh = jnp.maximum(lane_total(ah), EPS)
                nt = jnp.maximum(lane_total(at), EPS)
                nhnt = nh * nt
                os = [hs[c] * nt + rs[c] * nhnt - ts[c] * nh for c in range(D // L)]
                ao = (jnp.abs(os[0]) + jnp.abs(os[1])) + (jnp.abs(os[2]) + jnp.abs(os[3]))
                inv_o = 1.0 / jnp.maximum(lane_total(ao), EPS)
                for c in range(D // L):
                    o_v[i, pl.ds(c * L, L)] = os[c] * inv_o
                return carry2

            lax.fori_loop(0, C, row, 0)
            pltpu.sync_copy(o_v, out_hbm.at[pl.ds(base + cbase, C)])
            return carry

        lax.fori_loop(0, bpw // C, chunk, 0)

    return trans_e(head_index, rel_type, tail_index, node_rm, rel_emb)


# speedup vs baseline: 2.9409x; 2.9409x over previous
import functools

import jax
import jax.numpy as jnp
from jax import lax
from jax.experimental import pallas as pl
from jax.experimental.pallas import tpu as pltpu
from jax.experimental.pallas import tpu_sc as plsc

B = 16384
D = 64
L = 16
C = 128
EPS = 1e-12


def kernel(head_index, rel_type, tail_index, node_emb, rel_emb):
    info = plsc.get_sparse_core_info()
    nw = info.num_cores * info.num_subcores
    bpw = B // nw

    mesh = plsc.VectorSubcoreMesh(core_axis_name="c", subcore_axis_name="s")

    @functools.partial(
        pl.kernel,
        mesh=mesh,
        out_type=jax.ShapeDtypeStruct((B, D), jnp.float32),
        scratch_types=[
            pltpu.VMEM((bpw,), jnp.int32),
            pltpu.VMEM((bpw,), jnp.int32),
            pltpu.VMEM((bpw,), jnp.int32),
            pltpu.VMEM((C, D), jnp.float32),
            pltpu.VMEM((C, D), jnp.float32),
            pltpu.VMEM((C, D), jnp.float32),
            pltpu.VMEM((C, D), jnp.float32),
            pltpu.SemaphoreType.DMA,
        ],
    )
    def trans_e(h_idx_hbm, r_idx_hbm, t_idx_hbm, node_hbm, rel_hbm, out_hbm,
                hi_v, ri_v, ti_v, h_v, r_v, t_v, o_v, sem):
        wid = lax.axis_index("s") * info.num_cores + lax.axis_index("c")
        base = wid * bpw

        pltpu.sync_copy(h_idx_hbm.at[pl.ds(base, bpw)], hi_v)
        pltpu.sync_copy(r_idx_hbm.at[pl.ds(base, bpw)], ri_v)
        pltpu.sync_copy(t_idx_hbm.at[pl.ds(base, bpw)], ti_v)

        iota = lax.iota(jnp.int32, L)
        perms = [iota ^ sh for sh in (1, 2, 4, 8)]
        gdn = lax.GatherDimensionNumbers(
            offset_dims=(), collapsed_slice_dims=(0,), start_index_map=(0,))

        def lane_total(v):
            for p in perms:
                v = v + lax.gather(
                    v, p[:, None], dimension_numbers=gdn, slice_sizes=(1,),
                    mode=lax.GatherScatterMode.PROMISE_IN_BOUNDS)
            return v

        def chunk(ci, carry):
            cbase = ci * C
            copies = []
            for jj in range(C // L):
                hv = hi_v[pl.ds(cbase + jj * L, L)]
                tv = ti_v[pl.ds(cbase + jj * L, L)]
                rv = ri_v[pl.ds(cbase + jj * L, L)]
                for k in range(L):
                    r = jj * L + k
                    copies.append(pltpu.async_copy(
                        node_hbm.at[hv[k]], h_v.at[r], sem))
                    copies.append(pltpu.async_copy(
                        node_hbm.at[tv[k]], t_v.at[r], sem))
                    copies.append(pltpu.async_copy(
                        rel_hbm.at[rv[k]], r_v.at[r], sem))
            for cp in copies:
                cp.wait()

            def row(i, carry2):
                hs = [h_v[i, pl.ds(c * L, L)] for c in range(D // L)]
                ts = [t_v[i, pl.ds(c * L, L)] for c in range(D // L)]
                rs = [r_v[i, pl.ds(c * L, L)] for c in range(D // L)]

                ah = (jnp.abs(hs[0]) + jnp.abs(hs[1])) + (jnp.abs(hs[2]) + jnp.abs(hs[3]))
                at = (jnp.abs(ts[0]) + jnp.abs(ts[1])) + (jnp.abs(ts[2]) + jnp.abs(ts[3]))
                nh = jnp.maximum(lane_total(ah), EPS)
                nt = jnp.maximum(lane_total(at), EPS)
                nhnt = nh * nt
                os = [hs[c] * nt + rs[c] * nhnt - ts[c] * nh for c in range(D // L)]
                ao = (jnp.abs(os[0]) + jnp.abs(os[1])) + (jnp.abs(os[2]) + jnp.abs(os[3]))
                inv_o = 1.0 / jnp.maximum(lane_total(ao), EPS)
                for c in range(D // L):
                    o_v[i, pl.ds(c * L, L)] = os[c] * inv_o
                return carry2

            lax.fori_loop(0, C, row, 0)
            pltpu.sync_copy(o_v, out_hbm.at[pl.ds(base + cbase, C)])
            return carry

        lax.fori_loop(0, bpw // C, chunk, 0)

    return trans_e(head_index, rel_type, tail_index, node_emb, rel_emb)
